# baseline (speedup 1.0000x reference)
import functools

import jax
import jax.numpy as jnp
from jax import lax
from jax.experimental import pallas as pl
from jax.experimental.pallas import tpu as pltpu
from jax.experimental.pallas import tpu_sc as plsc

N_VALUES = 1000
LANES = 16
CHUNK = 128


@functools.cache
def _make_gather(B, V, D):
    info = plsc.get_sparse_core_info()
    nw = info.num_cores * info.num_subcores
    b_per_w = B // nw
    n_chunks = b_per_w // CHUNK
    mesh = plsc.VectorSubcoreMesh(core_axis_name="c", subcore_axis_name="s")

    @functools.partial(
        pl.kernel,
        mesh=mesh,
        out_type=jax.ShapeDtypeStruct((B, D), jnp.int32),
        compiler_params=pltpu.CompilerParams(use_tc_tiling_on_sc=False),
        scratch_types=[
            pltpu.VMEM((b_per_w,), jnp.int32),
            pltpu.VMEM((b_per_w,), jnp.int32),
            pltpu.VMEM((n_chunks, CHUNK), jnp.int32),
            pltpu.VMEM((b_per_w, D), jnp.int32),
            pltpu.SemaphoreType.DMA,
        ],
    )
    def gather_kernel(x_hbm, sh_hbm, table_hbm, out_hbm, x_v, sh_v, idx_v, rows_v, sem):
        wid = lax.axis_index("s") * info.num_cores + lax.axis_index("c")
        base = wid * b_per_w
        pltpu.sync_copy(x_hbm.at[pl.ds(base, b_per_w)], x_v)
        pltpu.sync_copy(sh_hbm.at[pl.ds(base, b_per_w)], sh_v)
        for c in range(n_chunks):
            for v in range(CHUNK // LANES):
                sl = pl.ds(c * CHUNK + v * LANES, LANES)
                idx_v[c, pl.ds(v * LANES, LANES)] = (
                    x_v[sl] * N_VALUES + sh_v[sl])
        copies = []
        for c in range(n_chunks):
            cp = pltpu.make_async_copy(
                table_hbm.at[idx_v.at[c]],
                rows_v.at[pl.ds(c * CHUNK, CHUNK)], sem)
            cp.start()
            copies.append(cp)
        for cp in copies:
            cp.wait()
        pltpu.sync_copy(rows_v, out_hbm.at[pl.ds(base, b_per_w)])

    return gather_kernel


def kernel(x, mapping):
    batch = x.shape[0]
    V, D = mapping.shape
    key = jax.random.key(42)
    shifts = jnp.stack(
        [jax.random.randint(jax.random.fold_in(key, i), (batch,), 0, N_VALUES,
                            dtype=x.dtype) for i in range(2)],
        axis=1)
    table8 = jnp.pad(mapping, ((0, 0), (0, 8 - D)))
    gathered = _make_gather(2 * batch, V, 8)(
        x.reshape(-1), shifts.reshape(-1), table8)
    result = gathered.reshape(batch, 2, 8)[:, :, :D].reshape(batch, 2 * D) + 1
    zeros = jnp.zeros(result.shape, jnp.float32)
    return (result, zeros, zeros)


# speedup vs baseline: 1.7149x; 1.7149x over previous
import functools

import jax
import jax.numpy as jnp
from jax import lax
from jax.experimental import pallas as pl
from jax.experimental.pallas import tpu as pltpu
from jax.experimental.pallas import tpu_sc as plsc

N_VALUES = 1000
LANES = 16
CHUNK = 128


@functools.cache
def _make_gather(B, V, D):
    info = plsc.get_sparse_core_info()
    nw = info.num_cores * info.num_subcores
    b_per_w = B // nw
    n_chunks = b_per_w // CHUNK
    mesh = plsc.VectorSubcoreMesh(core_axis_name="c", subcore_axis_name="s")

    @functools.partial(
        pl.kernel,
        mesh=mesh,
        out_type=jax.ShapeDtypeStruct((B, D), jnp.int32),
        compiler_params=pltpu.CompilerParams(use_tc_tiling_on_sc=False),
        scratch_types=[
            pltpu.VMEM((b_per_w,), jnp.int32),
            pltpu.VMEM((b_per_w,), jnp.int32),
            pltpu.VMEM((n_chunks, CHUNK), jnp.int32),
            pltpu.VMEM((b_per_w, D), jnp.int32),
            pltpu.SemaphoreType.DMA,
        ],
    )
    def gather_kernel(x_hbm, sh_hbm, table_hbm, out_hbm, x_v, sh_v, idx_v, rows_v, sem):
        wid = lax.axis_index("s") * info.num_cores + lax.axis_index("c")
        base = wid * b_per_w
        pltpu.sync_copy(x_hbm.at[pl.ds(base, b_per_w)], x_v)
        pltpu.sync_copy(sh_hbm.at[pl.ds(base, b_per_w)], sh_v)
        for c in range(n_chunks):
            for v in range(CHUNK // LANES):
                sl = pl.ds(c * CHUNK + v * LANES, LANES)
                idx_v[c, pl.ds(v * LANES, LANES)] = (
                    x_v[sl] * N_VALUES + sh_v[sl])
        copies = []
        for c in range(n_chunks):
            cp = pltpu.make_async_copy(
                table_hbm.at[idx_v.at[c]],
                rows_v.at[pl.ds(c * CHUNK, CHUNK)], sem)
            cp.start()
            copies.append(cp)
        for cp in copies:
            cp.wait()
        pltpu.sync_copy(rows_v, out_hbm.at[pl.ds(base, b_per_w)])

    return gather_kernel


def kernel(x, mapping):
    batch = x.shape[0]
    V, D = mapping.shape
    key = jax.random.key(42)
    shifts = jnp.stack(
        [jax.random.randint(jax.random.fold_in(key, i), (batch,), 0, N_VALUES,
                            dtype=x.dtype) for i in range(2)],
        axis=1)
    nt = -(-V // 128)
    vp = nt * 128
    table8 = (
        jnp.pad(mapping.T, ((0, 8 - D), (0, vp - V)))
        .reshape(8, nt, 128)
        .transpose(1, 2, 0)
        .reshape(nt * 8, 128)
        .reshape(vp, 8))
    gathered = _make_gather(2 * batch, vp, 8)(
        x.reshape(-1), shifts.reshape(-1), table8)
    result = gathered.reshape(batch, 2, 8)[:, :, :D].reshape(batch, 2 * D) + 1
    zeros = jnp.zeros(result.shape, jnp.float32)
    return (result, zeros, zeros)
